# baseline (speedup 1.0000x reference)
import functools
import math

import jax
import jax.numpy as jnp
from jax import lax
from jax.experimental import pallas as pl
from jax.experimental.pallas import tpu as pltpu
from jax.experimental.pallas import tpu_sc as plsc

D_MODEL = 64
_SCALE = math.sqrt(D_MODEL)
_CHUNK = 160
_NBUF = 4
_LOOK = 3

_SC_PARAMS = pltpu.CompilerParams(
    use_tc_tiling_on_sc=True,
    needs_layout_passes=False,
    skip_device_barrier=True,
    disable_semaphore_checks=True,
    disable_bounds_checks=True,
)


@functools.lru_cache(maxsize=None)
def _make_gather_kernel(n_rows: int):
    info = plsc.get_sparse_core_info()
    num_workers = info.num_cores * info.num_subcores
    rows_per_worker = n_rows // num_workers
    assert rows_per_worker * num_workers == n_rows
    n_chunks = rows_per_worker // _CHUNK
    assert n_chunks * _CHUNK == rows_per_worker
    assert n_chunks % _NBUF == 0 and n_chunks >= 2 * _NBUF

    mesh = plsc.VectorSubcoreMesh(core_axis_name="c", subcore_axis_name="s")

    @functools.partial(
        pl.kernel,
        mesh=mesh,
        out_type=jax.ShapeDtypeStruct((n_rows, D_MODEL), jnp.float32),
        scratch_types=(
            [pltpu.VMEM((_CHUNK,), jnp.int32) for _ in range(_NBUF)]
            + [pltpu.VMEM((_CHUNK, 128), jnp.float32) for _ in range(_NBUF)]
            + [pltpu.VMEM((_CHUNK, D_MODEL), jnp.float32) for _ in range(2)]
            + [pltpu.SemaphoreType.DMA for _ in range(_NBUF + 2)]
        ),
        compiler_params=_SC_PARAMS,
    )
    def gather_kernel(x_hbm, tab_hbm, out_hbm, *scratch):
        idx_bufs = scratch[:_NBUF]
        rows_bufs = scratch[_NBUF : 2 * _NBUF]
        out_bufs = scratch[2 * _NBUF : 2 * _NBUF + 2]
        gsems = scratch[2 * _NBUF + 2 : 3 * _NBUF + 2]
        osems = scratch[3 * _NBUF + 2 : 3 * _NBUF + 4]

        wid = lax.axis_index("s") * info.num_cores + lax.axis_index("c")
        base = wid * rows_per_worker

        def issue_gather(chunk, b):
            off = base + chunk * _CHUNK
            pltpu.sync_copy(x_hbm.at[pl.ds(off, _CHUNK)], idx_bufs[b])
            pltpu.async_copy(tab_hbm.at[idx_bufs[b]], rows_bufs[b], gsems[b])

        def wait_gather(b):
            pltpu.make_async_copy(
                tab_hbm.at[idx_bufs[b]], rows_bufs[b], gsems[b]
            ).wait()

        def issue_out(chunk, bo):
            off = base + chunk * _CHUNK
            pltpu.async_copy(
                out_bufs[bo], out_hbm.at[pl.ds(off, _CHUNK)], osems[bo]
            )

        def wait_out(bo):
            pltpu.make_async_copy(
                out_bufs[bo], out_hbm.at[pl.ds(0, _CHUNK)], osems[bo]
            ).wait()

        def scale_pass(b, bo):
            @plsc.parallel_loop(0, _CHUNK, unroll=8)
            def rbody(r):
                for k in range(D_MODEL // 16):
                    sl = (r, pl.ds(k * 16, 16))
                    out_bufs[bo][sl] = rows_bufs[b][sl] * _SCALE

        for i in range(_LOOK):
            issue_gather(i, i)

        def outer(it, carry):
            for b in range(_NBUF):
                i = it * _NBUF + b
                j = i + _LOOK
                bj = (b + _LOOK) % _NBUF
                bo = b % 2

                @pl.when(j < n_chunks)
                def _():
                    issue_gather(j, bj)

                wait_gather(b)

                @pl.when(i >= 2)
                def _():
                    wait_out(bo)

                scale_pass(b, bo)
                issue_out(i, bo)
            return carry

        lax.fori_loop(0, n_chunks // _NBUF, outer, 0)

        for bo in range(2):
            wait_out(bo)

    return gather_kernel


def kernel(x, lut):
    batch, seq = x.shape
    flat = x.reshape(batch * seq)
    tab = jnp.pad(lut, ((0, 0), (0, 128 - D_MODEL)))
    out = _make_gather_kernel(batch * seq)(flat, tab)
    return out.reshape(batch, seq, D_MODEL)
